# baseline (device time: 40357 ns/iter reference)
import functools

import jax
import jax.numpy as jnp
from jax import lax
from jax.experimental import pallas as pl
from jax.experimental.pallas import tpu as pltpu

N_DEV = 8
NC = 8
XOR_ROUNDS = (1, 3, 4)


def kernel(x, Wp):
    b, h_loc, w, c = x.shape
    c_out = Wp.shape[1]
    n_global = float(h_loc * N_DEV * w)
    ch = h_loc // NC

    def body(x_ref, wp_ref, out_ref, x_vmem, out_vmem, send_buf, recv_buf,
             load_sems, store_sems, send_sems, recv_sems):
        my_pos = lax.axis_index("i")

        loads = []
        for k in range(NC):
            cp = pltpu.make_async_copy(
                x_ref.at[:, pl.ds(k * ch, ch)],
                x_vmem.at[:, pl.ds(k * ch, ch)],
                load_sems.at[k],
            )
            cp.start()
            loads.append(cp)

        barrier_sem = pltpu.get_barrier_semaphore()
        for k in XOR_ROUNDS:
            pl.semaphore_signal(
                barrier_sem, inc=1,
                device_id=(jnp.bitwise_xor(my_pos, k),),
                device_id_type=pltpu.DeviceIdType.MESH,
            )
        pl.semaphore_wait(barrier_sem, len(XOR_ROUNDS))

        parts = []
        for k in range(NC):
            loads[k].wait()
            xa = x_vmem[:, k * ch:(k + 1) * ch]
            ps = jnp.sum(xa, axis=(1, 2))
            pss = jnp.sum(xa * xa, axis=(1, 2))
            parts.append(jnp.concatenate([ps, pss], axis=0))
        acc = functools.reduce(lambda u, v: u + v, parts)

        sends = []
        for r, k in enumerate(XOR_ROUNDS):
            partner = jnp.bitwise_xor(my_pos, k)
            send_buf[r] = acc
            rdma = pltpu.make_async_remote_copy(
                src_ref=send_buf.at[r],
                dst_ref=recv_buf.at[r],
                send_sem=send_sems.at[r],
                recv_sem=recv_sems.at[r],
                device_id=(partner,),
                device_id_type=pltpu.DeviceIdType.MESH,
            )
            rdma.start()
            rdma.wait_recv()
            sends.append(rdma)
            acc = acc + recv_buf[r]
        for rdma in sends:
            rdma.wait_send()

        mean = acc[:b] / n_global
        ex2 = acc[b:] / n_global
        var = ex2 - mean * mean
        inv = lax.rsqrt(var + 1e-5)
        mean_b = mean[:, None, None, :]
        inv_b = inv[:, None, None, :]
        wp16 = wp_ref[...].astype(jnp.bfloat16)

        stores = []
        for k in range(NC):
            slot = k % 2
            if k >= 2:
                stores[k - 2].wait()
            xa = x_vmem[:, k * ch:(k + 1) * ch]
            h = (xa - mean_b) * inv_b
            a = h * jax.nn.sigmoid(h)
            o = jnp.dot(
                a.reshape(b * ch * w, c).astype(jnp.bfloat16),
                wp16,
                preferred_element_type=jnp.float32,
            )
            out_vmem[slot] = o.reshape(b, ch, w, c_out)
            cp = pltpu.make_async_copy(
                out_vmem.at[slot],
                out_ref.at[:, pl.ds(k * ch, ch)],
                store_sems.at[slot],
            )
            cp.start()
            stores.append(cp)
        stores[-2].wait()
        stores[-1].wait()

    return pl.pallas_call(
        body,
        out_shape=jax.ShapeDtypeStruct((b, h_loc, w, c_out), jnp.float32),
        in_specs=[
            pl.BlockSpec(memory_space=pl.ANY),
            pl.BlockSpec(memory_space=pltpu.VMEM),
        ],
        out_specs=pl.BlockSpec(memory_space=pl.ANY),
        scratch_shapes=[
            pltpu.VMEM((b, h_loc, w, c), jnp.float32),
            pltpu.VMEM((2, b, ch, w, c_out), jnp.float32),
            pltpu.VMEM((len(XOR_ROUNDS), 2 * b, c), jnp.float32),
            pltpu.VMEM((len(XOR_ROUNDS), 2 * b, c), jnp.float32),
            pltpu.SemaphoreType.DMA((NC,)),
            pltpu.SemaphoreType.DMA((2,)),
            pltpu.SemaphoreType.DMA((len(XOR_ROUNDS),)),
            pltpu.SemaphoreType.DMA((len(XOR_ROUNDS),)),
        ],
        compiler_params=pltpu.CompilerParams(collective_id=0),
    )(x, Wp)
